# baseline (speedup 1.0000x reference)
import functools

import jax
import jax.numpy as jnp
from jax import lax
from jax.experimental import pallas as pl
from jax.experimental.pallas import tpu as pltpu
from jax.experimental.pallas import tpu_sc as plsc

N = 10000
E = 320000
D_IN = 128
DH = 64

NC = 2
NS = 16
NW = NC * NS

N_PAD = 10240
E_PAD = 327680
ROWS = E_PAD // 128
RPW = ROWS // NW
RPT = N_PAD // NS
NB = 2
DW = 16

@functools.lru_cache(maxsize=None)
def _sc_kernels():
  mesh = plsc.VectorSubcoreMesh(
      core_axis_name="c", subcore_axis_name="s", num_cores=NC)
  params = pltpu.CompilerParams(use_tc_tiling_on_sc=False)

  @functools.partial(
      pl.kernel,
      mesh=mesh,
      out_type=jax.ShapeDtypeStruct((NC, N_PAD, DW), jnp.float32),
      compiler_params=params,
      scratch_types=[
          pltpu.VMEM((RPW, 128), jnp.int32),
          pltpu.VMEM((128, DW), jnp.float32),
          pltpu.VMEM((128, DW), jnp.float32),
          pltpu.VMEM_SHARED((N_PAD, DW), jnp.float32),
      ],
  )
  def _deg_sc(dst_hbm, ones_hbm, zeros_hbm, out_hbm, dst_v, ones_v, zero_v,
              dacc):
    c = lax.axis_index("c")
    s = lax.axis_index("s")
    w = c * NS + s

    pltpu.sync_copy(ones_hbm, ones_v)
    pltpu.sync_copy(zeros_hbm, zero_v)
    pltpu.sync_copy(dst_hbm.at[pl.ds(w * RPW, RPW)], dst_v)
    for k in range(RPT // 128):
      pltpu.sync_copy(zero_v, dacc.at[pl.ds(s * RPT + k * 128, 128)])
    plsc.subcore_barrier()

    def body(j, _):
      pltpu.sync_copy(ones_v, dacc.at[dst_v.at[j]], add=True)
      return _

    lax.fori_loop(0, RPW, body, 0)
    plsc.subcore_barrier()
    pltpu.sync_copy(dacc.at[pl.ds(s * RPT, RPT)],
                    out_hbm.at[c, pl.ds(s * RPT, RPT)])

  @functools.partial(
      pl.kernel,
      mesh=mesh,
      out_type=jax.ShapeDtypeStruct((NC, N_PAD, DH), jnp.float32),
      compiler_params=params,
      scratch_types=[
          pltpu.VMEM((RPW, 128), jnp.int32),
          pltpu.VMEM((RPW, 128), jnp.int32),
          [pltpu.VMEM((128, DH), jnp.float32) for _ in range(NB)],
          pltpu.VMEM_SHARED((N_PAD, DH), jnp.float32),
          pltpu.VMEM_SHARED((N_PAD, DH), jnp.float32),
          [pltpu.SemaphoreType.DMA for _ in range(NB)],
          [pltpu.SemaphoreType.DMA for _ in range(NB)],
      ],
  )
  def _prop_sc(hp_hbm, src_hbm, dst_hbm, out_hbm, src_v, dst_v, bufs,
               acc, tbl, gsems, ssems):
    c = lax.axis_index("c")
    s = lax.axis_index("s")

    for k in range(RPT // 128):
      pltpu.sync_copy(hp_hbm.at[pl.ds(s * RPT + k * 128, 128)], bufs[0])
      pltpu.sync_copy(bufs[0], acc.at[pl.ds(s * RPT + k * 128, 128)])

    for k in range(RPT // 128):
      pltpu.sync_copy(hp_hbm.at[pl.ds(s * RPT + k * 128, 128)], bufs[1])
      pltpu.sync_copy(bufs[1], tbl.at[pl.ds(s * RPT + k * 128, 128)])

    def ring(nrows, row0, table):
      pltpu.sync_copy(src_hbm.at[pl.ds(row0, nrows)], src_v.at[pl.ds(0, nrows)])
      pltpu.sync_copy(dst_hbm.at[pl.ds(row0, nrows)], dst_v.at[pl.ds(0, nrows)])
      plsc.subcore_barrier()
      for k in range(NB):
        pltpu.async_copy(table.at[src_v.at[k]], bufs[k], gsems[k])

      def body(i, carry):
        j = NB * i
        for k in range(NB):
          pltpu.make_async_copy(
              table.at[src_v.at[j + k]], bufs[k], gsems[k]).wait()
          pltpu.async_copy(bufs[k], acc.at[dst_v.at[j + k]], ssems[k],
                           add=True)
        for k in range(NB):
          @pl.when(j + k + NB < nrows)
          def _(k=k, j=j):
            pltpu.make_async_copy(
                bufs[k], acc.at[dst_v.at[j + k]], ssems[k]).wait()
            pltpu.async_copy(table.at[src_v.at[j + k + NB]], bufs[k],
                             gsems[k])
        return carry

      lax.fori_loop(0, nrows // NB, body, 0)
      for k in range(NB):
        pltpu.make_async_copy(
            bufs[k], acc.at[dst_v.at[nrows - NB + k]], ssems[k]).wait()

    ring(RPW, (c * NS + s) * RPW, tbl)
    plsc.subcore_barrier()
    pltpu.sync_copy(acc.at[pl.ds(s * RPT, RPT)],
                    out_hbm.at[c, pl.ds(s * RPT, RPT)])

  return _deg_sc, _prop_sc


def _dense1_body(x_ref, w1_ref, degp_ref, hp_ref, dis_ref):
    deg = degp_ref[0, :, 0:1] + degp_ref[1, :, 0:1] + 1.0
    dis = lax.rsqrt(deg)
    h = jnp.dot(x_ref[...], w1_ref[...], preferred_element_type=jnp.float32)
    hp = h * dis[0:N, :]
    hp_ref[...] = jnp.concatenate(
        [hp, jnp.zeros((N_PAD - N, DH), jnp.float32)], axis=0)
    dis_ref[...] = dis


def _dense2_body(p_ref, hp_ref, dis_ref, b_ref, w_ref, out_ref):
    p = p_ref[0] + p_ref[1]
    pre = (p - hp_ref[...]) * dis_ref[...] + b_ref[...]
    o = jnp.maximum(pre, 0.0)
    out_ref[...] = jnp.dot(o, w_ref[...],
                           preferred_element_type=jnp.float32) * dis_ref[...]


def _dense3_body(p_ref, hp_ref, dis_ref, b_ref, wfc_ref, bfc_ref, y_ref):
    p = p_ref[0] + p_ref[1]
    pre = (p - hp_ref[...]) * dis_ref[...] + b_ref[...]
    o = jnp.maximum(pre, 0.0)
    z = jnp.dot(o, wfc_ref[...], preferred_element_type=jnp.float32) + bfc_ref[...]
    z = z[0:N, :]
    y_ref[...] = 1.0 / (1.0 + jnp.exp(-z))


_dense1 = pl.pallas_call(
    _dense1_body,
    out_shape=[
        jax.ShapeDtypeStruct((N_PAD, DH), jnp.float32),
        jax.ShapeDtypeStruct((N_PAD, 1), jnp.float32),
    ],
)

_dense2 = pl.pallas_call(
    _dense2_body,
    out_shape=jax.ShapeDtypeStruct((N_PAD, DH), jnp.float32),
)

_dense3 = pl.pallas_call(
    _dense3_body,
    out_shape=jax.ShapeDtypeStruct((N, 1), jnp.float32),
)


def kernel(x, edge_index, W1, b1, W2, b2, Wfc, bfc):
    src = edge_index[0]
    dst = edge_index[1]
    pad = E_PAD - E
    src_r = jnp.concatenate(
        [src, jnp.zeros((pad,), jnp.int32)]).reshape(ROWS, 128)
    dst_r = jnp.concatenate(
        [dst, jnp.full((pad,), N, jnp.int32)]).reshape(ROWS, 128)

    _deg_sc, _prop_sc = _sc_kernels()
    ones_c = jnp.ones((128, DW), jnp.float32)
    zeros_c = jnp.zeros((128, DW), jnp.float32)
    degp = _deg_sc(dst_r, ones_c, zeros_c)
    hp1, dis = _dense1(x, W1, degp)
    p1 = _prop_sc(hp1, src_r, dst_r)
    hp2 = _dense2(p1, hp1, dis, b1.reshape(1, DH), W2)
    p2 = _prop_sc(hp2, src_r, dst_r)
    return _dense3(p2, hp2, dis, b2.reshape(1, DH), Wfc, bfc.reshape(1, 1))


# speedup vs baseline: 1.0065x; 1.0065x over previous
import functools

import jax
import jax.numpy as jnp
from jax import lax
from jax.experimental import pallas as pl
from jax.experimental.pallas import tpu as pltpu
from jax.experimental.pallas import tpu_sc as plsc

N = 10000
E = 320000
D_IN = 128
DH = 64

NC = 2
NS = 16
NW = NC * NS

N_PAD = 10240
E_PAD = 327680
ROWS = E_PAD // 128
RPW = ROWS // NW
RPT = N_PAD // NS
NB = 2
DW = 16

@functools.lru_cache(maxsize=None)
def _sc_kernels():
  mesh = plsc.VectorSubcoreMesh(
      core_axis_name="c", subcore_axis_name="s", num_cores=NC)
  params = pltpu.CompilerParams(use_tc_tiling_on_sc=False)

  @functools.partial(
      pl.kernel,
      mesh=mesh,
      out_type=jax.ShapeDtypeStruct((NC, N_PAD, DW), jnp.float32),
      compiler_params=params,
      scratch_types=[
          pltpu.VMEM((RPW, 128), jnp.int32),
          pltpu.VMEM((128, DW), jnp.float32),
          pltpu.VMEM((128, DW), jnp.float32),
          pltpu.VMEM_SHARED((N_PAD, DW), jnp.float32),
      ],
  )
  def _deg_sc(dst_hbm, out_hbm, dst_v, ones_v, zero_v, dacc):
    c = lax.axis_index("c")
    s = lax.axis_index("s")
    w = c * NS + s

    def fill(i, _):
      ones_v[i] = jnp.full((16,), 1.0, jnp.float32)
      zero_v[i] = jnp.zeros((16,), jnp.float32)
      return _

    lax.fori_loop(0, 128, fill, 0)
    pltpu.sync_copy(dst_hbm.at[pl.ds(w * RPW, RPW)], dst_v)
    for k in range(RPT // 128):
      pltpu.sync_copy(zero_v, dacc.at[pl.ds(s * RPT + k * 128, 128)])
    plsc.subcore_barrier()

    def body(j, _):
      pltpu.sync_copy(ones_v, dacc.at[dst_v.at[j]], add=True)
      return _

    lax.fori_loop(0, RPW, body, 0)
    plsc.subcore_barrier()
    pltpu.sync_copy(dacc.at[pl.ds(s * RPT, RPT)],
                    out_hbm.at[c, pl.ds(s * RPT, RPT)])

  @functools.partial(
      pl.kernel,
      mesh=mesh,
      out_type=jax.ShapeDtypeStruct((NC, N_PAD, DH), jnp.float32),
      compiler_params=params,
      scratch_types=[
          pltpu.VMEM((RPW, 128), jnp.int32),
          pltpu.VMEM((RPW, 128), jnp.int32),
          [pltpu.VMEM((128, DH), jnp.float32) for _ in range(NB)],
          pltpu.VMEM_SHARED((N_PAD, DH), jnp.float32),
          pltpu.VMEM_SHARED((N_PAD, DH), jnp.float32),
          [pltpu.SemaphoreType.DMA for _ in range(NB)],
          [pltpu.SemaphoreType.DMA for _ in range(NB)],
      ],
  )
  def _prop_sc(hp_hbm, src_hbm, dst_hbm, out_hbm, src_v, dst_v, bufs,
               acc, tbl, gsems, ssems):
    c = lax.axis_index("c")
    s = lax.axis_index("s")

    for k in range(RPT // 128):
      pltpu.sync_copy(hp_hbm.at[pl.ds(s * RPT + k * 128, 128)], bufs[0])
      pltpu.sync_copy(bufs[0], acc.at[pl.ds(s * RPT + k * 128, 128)])

    for k in range(RPT // 128):
      pltpu.sync_copy(hp_hbm.at[pl.ds(s * RPT + k * 128, 128)], bufs[1])
      pltpu.sync_copy(bufs[1], tbl.at[pl.ds(s * RPT + k * 128, 128)])

    def ring(nrows, row0, table):
      pltpu.sync_copy(src_hbm.at[pl.ds(row0, nrows)], src_v.at[pl.ds(0, nrows)])
      pltpu.sync_copy(dst_hbm.at[pl.ds(row0, nrows)], dst_v.at[pl.ds(0, nrows)])
      plsc.subcore_barrier()
      for k in range(NB):
        pltpu.async_copy(table.at[src_v.at[k]], bufs[k], gsems[k])

      def body(i, carry):
        j = NB * i
        for k in range(NB):
          pltpu.make_async_copy(
              table.at[src_v.at[j + k]], bufs[k], gsems[k]).wait()
          pltpu.async_copy(bufs[k], acc.at[dst_v.at[j + k]], ssems[k],
                           add=True)
        for k in range(NB):
          @pl.when(j + k + NB < nrows)
          def _(k=k, j=j):
            pltpu.make_async_copy(
                bufs[k], acc.at[dst_v.at[j + k]], ssems[k]).wait()
            pltpu.async_copy(table.at[src_v.at[j + k + NB]], bufs[k],
                             gsems[k])
        return carry

      lax.fori_loop(0, nrows // NB, body, 0)
      for k in range(NB):
        pltpu.make_async_copy(
            bufs[k], acc.at[dst_v.at[nrows - NB + k]], ssems[k]).wait()

    ring(RPW, (c * NS + s) * RPW, tbl)
    plsc.subcore_barrier()
    pltpu.sync_copy(acc.at[pl.ds(s * RPT, RPT)],
                    out_hbm.at[c, pl.ds(s * RPT, RPT)])

  return _deg_sc, _prop_sc


def _dense1_body(x_ref, w1_ref, degp_ref, hp_ref, dis_ref):
    deg = degp_ref[0, :, 0:1] + degp_ref[1, :, 0:1] + 1.0
    dis = lax.rsqrt(deg)
    h = jnp.dot(x_ref[...], w1_ref[...], preferred_element_type=jnp.float32)
    hp = h * dis[0:N, :]
    hp_ref[...] = jnp.concatenate(
        [hp, jnp.zeros((N_PAD - N, DH), jnp.float32)], axis=0)
    dis_ref[...] = dis


def _dense2_body(p_ref, hp_ref, dis_ref, b_ref, w_ref, out_ref):
    p = p_ref[0] + p_ref[1]
    pre = (p - hp_ref[...]) * dis_ref[...] + b_ref[...]
    o = jnp.maximum(pre, 0.0)
    out_ref[...] = jnp.dot(o, w_ref[...],
                           preferred_element_type=jnp.float32) * dis_ref[...]


def _dense3_body(p_ref, hp_ref, dis_ref, b_ref, wfc_ref, bfc_ref, y_ref):
    p = p_ref[0] + p_ref[1]
    pre = (p - hp_ref[...]) * dis_ref[...] + b_ref[...]
    o = jnp.maximum(pre, 0.0)
    z = jnp.dot(o, wfc_ref[...], preferred_element_type=jnp.float32) + bfc_ref[...]
    z = z[0:N, :]
    y_ref[...] = 1.0 / (1.0 + jnp.exp(-z))


_dense1 = pl.pallas_call(
    _dense1_body,
    out_shape=[
        jax.ShapeDtypeStruct((N_PAD, DH), jnp.float32),
        jax.ShapeDtypeStruct((N_PAD, 1), jnp.float32),
    ],
)

_dense2 = pl.pallas_call(
    _dense2_body,
    out_shape=jax.ShapeDtypeStruct((N_PAD, DH), jnp.float32),
)

_dense3 = pl.pallas_call(
    _dense3_body,
    out_shape=jax.ShapeDtypeStruct((N, 1), jnp.float32),
)


def kernel(x, edge_index, W1, b1, W2, b2, Wfc, bfc):
    src = edge_index[0]
    dst = edge_index[1]
    pad = E_PAD - E
    src_r = jnp.concatenate(
        [src, jnp.zeros((pad,), jnp.int32)]).reshape(ROWS, 128)
    dst_r = jnp.concatenate(
        [dst, jnp.full((pad,), N, jnp.int32)]).reshape(ROWS, 128)

    _deg_sc, _prop_sc = _sc_kernels()
    degp = _deg_sc(dst_r)
    hp1, dis = _dense1(x, W1, degp)
    p1 = _prop_sc(hp1, src_r, dst_r)
    hp2 = _dense2(p1, hp1, dis, b1.reshape(1, DH), W2)
    p2 = _prop_sc(hp2, src_r, dst_r)
    return _dense3(p2, hp2, dis, b2.reshape(1, DH), Wfc, bfc.reshape(1, 1))


# speedup vs baseline: 1.0389x; 1.0322x over previous
import functools

import jax
import jax.numpy as jnp
from jax import lax
from jax.experimental import pallas as pl
from jax.experimental.pallas import tpu as pltpu
from jax.experimental.pallas import tpu_sc as plsc

N = 10000
E = 320000
D_IN = 128
DH = 64

NC = 2
NS = 16
NW = NC * NS

N_PAD = 10240
E_PAD = 327680
ROWS = E_PAD // 128
RPW = ROWS // NW
RPT = N_PAD // NS
NB = 2
DW = 16

@functools.lru_cache(maxsize=None)
def _sc_kernels():
  mesh = plsc.VectorSubcoreMesh(
      core_axis_name="c", subcore_axis_name="s", num_cores=NC)
  params = pltpu.CompilerParams(use_tc_tiling_on_sc=False)

  @functools.partial(
      pl.kernel,
      mesh=mesh,
      out_type=jax.ShapeDtypeStruct((NC, N_PAD, DW), jnp.float32),
      compiler_params=params,
      scratch_types=[
          pltpu.VMEM((RPW, 128), jnp.int32),
          pltpu.VMEM((128, DW), jnp.float32),
          pltpu.VMEM((128, DW), jnp.float32),
          pltpu.VMEM_SHARED((N_PAD, DW), jnp.float32),
      ],
  )
  def _deg_sc(dst_hbm, out_hbm, dst_v, ones_v, zero_v, dacc):
    c = lax.axis_index("c")
    s = lax.axis_index("s")
    w = c * NS + s

    def fill(i, _):
      ones_v[i] = jnp.full((16,), 1.0, jnp.float32)
      zero_v[i] = jnp.zeros((16,), jnp.float32)
      return _

    lax.fori_loop(0, 128, fill, 0)
    pltpu.sync_copy(dst_hbm.at[pl.ds(w * RPW, RPW)], dst_v)
    for k in range(RPT // 128):
      pltpu.sync_copy(zero_v, dacc.at[pl.ds(s * RPT + k * 128, 128)])
    plsc.subcore_barrier()

    def body(j, _):
      pltpu.sync_copy(ones_v, dacc.at[dst_v.at[j]], add=True)
      return _

    lax.fori_loop(0, RPW, body, 0)
    plsc.subcore_barrier()
    pltpu.sync_copy(dacc.at[pl.ds(s * RPT, RPT)],
                    out_hbm.at[c, pl.ds(s * RPT, RPT)])

  @functools.partial(
      pl.kernel,
      mesh=mesh,
      out_type=jax.ShapeDtypeStruct((NC, N_PAD, DH), jnp.float32),
      compiler_params=params,
      scratch_types=[
          pltpu.VMEM((RPW, 128), jnp.int32),
          pltpu.VMEM((RPW, 128), jnp.int32),
          [pltpu.VMEM((128, DH), jnp.float32) for _ in range(NB)],
          pltpu.VMEM_SHARED((N_PAD, DH), jnp.float32),
          pltpu.VMEM_SHARED((N_PAD, DH), jnp.float32),
          [pltpu.SemaphoreType.DMA for _ in range(NB)],
          [pltpu.SemaphoreType.DMA for _ in range(NB)],
      ],
  )
  def _prop_sc(hp_hbm, src_hbm, dst_hbm, out_hbm, src_v, dst_v, bufs,
               acc, tbl, gsems, ssems):
    c = lax.axis_index("c")
    s = lax.axis_index("s")

    w = c * NS + s

    for k in range(RPT // 128):
      pltpu.sync_copy(hp_hbm.at[pl.ds(s * RPT + k * 128, 128)], bufs[0])
      pltpu.sync_copy(bufs[0], acc.at[pl.ds(s * RPT + k * 128, 128)])
      pltpu.sync_copy(bufs[0], tbl.at[pl.ds(s * RPT + k * 128, 128)])
    pltpu.sync_copy(src_hbm.at[pl.ds(w * RPW, RPW)], src_v)
    pltpu.sync_copy(dst_hbm.at[pl.ds(w * RPW, RPW)], dst_v)
    plsc.subcore_barrier()

    for k in range(NB):
      pltpu.async_copy(tbl.at[src_v.at[k]], bufs[k], gsems[k])

    def body(i, carry):
      j = NB * i
      for k in range(NB):
        pltpu.make_async_copy(
            tbl.at[src_v.at[j + k]], bufs[k], gsems[k]).wait()
        pltpu.async_copy(bufs[k], acc.at[dst_v.at[j + k]], ssems[k], add=True)
      for k in range(NB):
        @pl.when(j + k + NB < RPW)
        def _(k=k, j=j):
          pltpu.make_async_copy(
              bufs[k], acc.at[dst_v.at[j + k]], ssems[k]).wait()
          pltpu.async_copy(tbl.at[src_v.at[j + k + NB]], bufs[k], gsems[k])
      return carry

    lax.fori_loop(0, RPW // NB, body, 0)
    for k in range(NB):
      pltpu.make_async_copy(
          bufs[k], acc.at[dst_v.at[RPW - NB + k]], ssems[k]).wait()
    plsc.subcore_barrier()
    pltpu.sync_copy(acc.at[pl.ds(s * RPT, RPT)],
                    out_hbm.at[c, pl.ds(s * RPT, RPT)])

  return _deg_sc, _prop_sc


def _dense1_body(x_ref, w1_ref, degp_ref, hp_ref, dis_ref):
    deg = degp_ref[0, :, 0:1] + degp_ref[1, :, 0:1] + 1.0
    dis = lax.rsqrt(deg)
    h = jnp.dot(x_ref[...], w1_ref[...], preferred_element_type=jnp.float32)
    hp = h * dis[0:N, :]
    hp_ref[...] = jnp.concatenate(
        [hp, jnp.zeros((N_PAD - N, DH), jnp.float32)], axis=0)
    dis_ref[...] = dis


def _dense2_body(p_ref, hp_ref, dis_ref, b_ref, w_ref, out_ref):
    p = p_ref[0] + p_ref[1]
    pre = (p - hp_ref[...]) * dis_ref[...] + b_ref[...]
    o = jnp.maximum(pre, 0.0)
    out_ref[...] = jnp.dot(o, w_ref[...],
                           preferred_element_type=jnp.float32) * dis_ref[...]


def _dense3_body(p_ref, hp_ref, dis_ref, b_ref, wfc_ref, bfc_ref, y_ref):
    p = p_ref[0] + p_ref[1]
    pre = (p - hp_ref[...]) * dis_ref[...] + b_ref[...]
    o = jnp.maximum(pre, 0.0)
    z = jnp.dot(o, wfc_ref[...], preferred_element_type=jnp.float32) + bfc_ref[...]
    z = z[0:N, :]
    y_ref[...] = 1.0 / (1.0 + jnp.exp(-z))


_dense1 = pl.pallas_call(
    _dense1_body,
    out_shape=[
        jax.ShapeDtypeStruct((N_PAD, DH), jnp.float32),
        jax.ShapeDtypeStruct((N_PAD, 1), jnp.float32),
    ],
)

_dense2 = pl.pallas_call(
    _dense2_body,
    out_shape=jax.ShapeDtypeStruct((N_PAD, DH), jnp.float32),
)

_dense3 = pl.pallas_call(
    _dense3_body,
    out_shape=jax.ShapeDtypeStruct((N, 1), jnp.float32),
)


def kernel(x, edge_index, W1, b1, W2, b2, Wfc, bfc):
    src = edge_index[0]
    dst = edge_index[1]
    pad = E_PAD - E
    src_r = jnp.concatenate(
        [src, jnp.zeros((pad,), jnp.int32)]).reshape(ROWS, 128)
    dst_r = jnp.concatenate(
        [dst, jnp.full((pad,), N, jnp.int32)]).reshape(ROWS, 128)

    _deg_sc, _prop_sc = _sc_kernels()
    degp = _deg_sc(dst_r)
    hp1, dis = _dense1(x, W1, degp)
    p1 = _prop_sc(hp1, src_r, dst_r)
    hp2 = _dense2(p1, hp1, dis, b1.reshape(1, DH), W2)
    p2 = _prop_sc(hp2, src_r, dst_r)
    return _dense3(p2, hp2, dis, b2.reshape(1, DH), Wfc, bfc.reshape(1, 1))
